# baseline (speedup 1.0000x reference)
import functools

import jax
import jax.numpy as jnp
from jax import lax
from jax.experimental import pallas as pl
from jax.experimental.pallas import tpu as pltpu
from jax.experimental.pallas import tpu_sc as plsc

B = 16384
N_DENSE = 13
N_SPARSE = 26
VOCAB = 100000
EMB = 32
N_CROSS = 3
OUT_DIM = 64
X_DIM = N_DENSE + N_SPARSE * EMB

_SC_CORES = 2
_SC_SUBCORES = 16
_NW = _SC_CORES * _SC_SUBCORES

_HALF_F = N_SPARSE // 2
_H_ROWS = B * _HALF_F
_PER_W = _H_ROWS // _NW
_IDXW = 128
_SLICES = 4
_CHUNK = _SLICES * _IDXW
_N_CHUNKS = _PER_W // _CHUNK


def _sc_gather(table_flat, flat_idx2d):
    mesh = plsc.VectorSubcoreMesh(core_axis_name="c", subcore_axis_name="s")

    @functools.partial(
        pl.kernel,
        mesh=mesh,
        compiler_params=pltpu.CompilerParams(use_tc_tiling_on_sc=False),
        out_type=jax.ShapeDtypeStruct((_H_ROWS, EMB), jnp.float32),
        scratch_types=[
            pltpu.VMEM((_SLICES, _IDXW), jnp.int32),
            pltpu.VMEM((_CHUNK, EMB), jnp.float32),
            pltpu.SemaphoreType.DMA,
        ],
    )
    def gather_k(table_hbm, idx_hbm, out_hbm, idx_v, rows_v, sem):
        wid = lax.axis_index("s") * _SC_CORES + lax.axis_index("c")
        base = wid * _PER_W

        def chunk_body(i, carry):
            off = base + i * _CHUNK
            pltpu.sync_copy(idx_hbm.at[pl.ds(off // _IDXW, _SLICES)], idx_v)
            for j in range(_SLICES):
                pltpu.async_copy(
                    table_hbm.at[idx_v.at[j]],
                    rows_v.at[pl.ds(j * _IDXW, _IDXW)], sem)
            for j in range(_SLICES):
                pltpu.make_async_copy(
                    table_hbm.at[idx_v.at[j]],
                    rows_v.at[pl.ds(j * _IDXW, _IDXW)], sem).wait()
            pltpu.sync_copy(rows_v, out_hbm.at[pl.ds(off, _CHUNK)])
            return carry

        lax.fori_loop(0, _N_CHUNKS, chunk_body, 0)

    return gather_k(table_flat, flat_idx2d)


_BT = 1024


def _dcn_block(inp_ref, embA_ref, embB_ref, cw_ref, b1_ref, w1_ref, w2_ref,
               b2_ref, w3_ref, b3_ref, wo_ref, sc_ref, out_ref):
    x = jnp.concatenate(
        [inp_ref[:, :N_DENSE]]
        + [embA_ref[f] for f in range(_HALF_F)]
        + [embB_ref[f] for f in range(_HALF_F)], axis=1)

    h = jnp.maximum(
        jnp.dot(x, w1_ref[...], preferred_element_type=jnp.float32)
        + b1_ref[...], 0.0)
    h = jnp.maximum(
        jnp.dot(h, w2_ref[...], preferred_element_type=jnp.float32)
        + b2_ref[...], 0.0)
    dnn = jnp.maximum(
        jnp.dot(h, w3_ref[...], preferred_element_type=jnp.float32)
        + b3_ref[...], 0.0)

    xl = x
    for i in range(N_CROSS):
        alpha = jnp.dot(xl, cw_ref[:, i:i + 1],
                        preferred_element_type=jnp.float32)
        xl = (x * alpha + sc_ref[:, i:i + 1]) + xl

    cat = jnp.concatenate([xl, dnn], axis=1)
    logit = jnp.dot(cat, wo_ref[...],
                    preferred_element_type=jnp.float32) + sc_ref[:, 3:4]
    out_ref[...] = jax.nn.sigmoid(logit)


def _tc_dcn(inputs, embA, embB, cw, b1, w1, w2, b2, w3, b3, wo, sc,
            interpret=False):
    grid = (B // _BT,)

    def full(shape):
        return pl.BlockSpec(shape, lambda i: tuple(0 for _ in shape))

    return pl.pallas_call(
        _dcn_block,
        grid=grid,
        in_specs=[
            pl.BlockSpec((_BT, N_DENSE + N_SPARSE), lambda i: (i, 0)),
            pl.BlockSpec((_HALF_F, _BT, EMB), lambda i: (0, i, 0)),
            pl.BlockSpec((_HALF_F, _BT, EMB), lambda i: (0, i, 0)),
            full(cw.shape),
            full(b1.shape),
            full(w1.shape),
            full(w2.shape),
            full(b2.shape),
            full(w3.shape),
            full(b3.shape),
            full(wo.shape),
            full(sc.shape),
        ],
        out_specs=pl.BlockSpec((_BT, 1), lambda i: (i, 0)),
        out_shape=jax.ShapeDtypeStruct((B, 1), jnp.float32),
        interpret=interpret,
    )(inputs, embA, embB, cw, b1, w1, w2, b2, w3, b3, wo, sc)


def kernel(inputs, embed_tables, cross_w, cross_b, W1, b1, W2, b2, W3, b3,
           Wo, bo):
    vT = inputs[:, N_DENSE:].astype(jnp.int32).T
    fl = jnp.arange(_HALF_F, dtype=jnp.int32)[:, None] * VOCAB
    embs = []
    for h in range(2):
        enc = (vT[h * _HALF_F:(h + 1) * _HALF_F] + fl
               ).reshape(_H_ROWS // _IDXW, _IDXW)
        tbl = embed_tables[h * _HALF_F:(h + 1) * _HALF_F
                           ].reshape(_HALF_F * VOCAB, EMB)
        embs.append(_sc_gather(tbl, enc).reshape(_HALF_F, B, EMB))

    cw = jnp.concatenate([cross_w[0], cross_w[1], cross_w[2]], axis=1)
    sc = jnp.concatenate([cross_b.reshape(-1), bo.reshape(-1)]).reshape(1, 4)
    return _tc_dcn(inputs, embs[0], embs[1], cw, b1.reshape(1, -1), W1,
                   W2, b2.reshape(1, -1), W3, b3.reshape(1, -1), Wo, sc)


# speedup vs baseline: 1.5491x; 1.5491x over previous
import functools

import jax
import jax.numpy as jnp
from jax import lax
from jax.experimental import pallas as pl
from jax.experimental.pallas import tpu as pltpu
from jax.experimental.pallas import tpu_sc as plsc

B = 16384
N_DENSE = 13
N_SPARSE = 26
VOCAB = 100000
EMB = 32
N_CROSS = 3
OUT_DIM = 64
X_DIM = N_DENSE + N_SPARSE * EMB

_SC_CORES = 2
_SC_SUBCORES = 16
_NW = _SC_CORES * _SC_SUBCORES

_N_ROWS = B * N_SPARSE
_PER_W = _N_ROWS // _NW
_IDXW = 128
_SLICES = 8
_CHUNK = _SLICES * _IDXW
_N_CHUNKS = _PER_W // _CHUNK


def _sc_gather(table_flat, flat_idx2d):
    mesh = plsc.VectorSubcoreMesh(core_axis_name="c", subcore_axis_name="s")

    @functools.partial(
        pl.kernel,
        mesh=mesh,
        compiler_params=pltpu.CompilerParams(use_tc_tiling_on_sc=False),
        out_type=jax.ShapeDtypeStruct((_N_ROWS, EMB), jnp.float32),
        scratch_types=[
            pltpu.VMEM((_SLICES, _IDXW), jnp.int32),
            pltpu.VMEM((_CHUNK, EMB), jnp.float32),
            pltpu.SemaphoreType.DMA,
        ],
    )
    def gather_k(table_hbm, idx_hbm, out_hbm, idx_v, rows_v, sem):
        wid = lax.axis_index("s") * _SC_CORES + lax.axis_index("c")
        base = wid * _PER_W

        def chunk_body(i, carry):
            off = base + i * _CHUNK
            pltpu.sync_copy(idx_hbm.at[pl.ds(off // _IDXW, _SLICES)], idx_v)
            for j in range(_SLICES):
                pltpu.async_copy(
                    table_hbm.at[idx_v.at[j]],
                    rows_v.at[pl.ds(j * _IDXW, _IDXW)], sem)
            for j in range(_SLICES):
                pltpu.make_async_copy(
                    table_hbm.at[idx_v.at[j]],
                    rows_v.at[pl.ds(j * _IDXW, _IDXW)], sem).wait()
            pltpu.sync_copy(rows_v, out_hbm.at[pl.ds(off, _CHUNK)])
            return carry

        lax.fori_loop(0, _N_CHUNKS, chunk_body, 0)

    return gather_k(table_flat, flat_idx2d)


_BT = 2048


def _dcn_block(inp_ref, emb_ref, cw_ref, b1_ref, w1_ref, w2_ref, b2_ref,
               w3_ref, b3_ref, wo_ref, sc_ref, out_ref):
    x = jnp.concatenate([inp_ref[:, :N_DENSE], emb_ref[...]], axis=1)

    h = jnp.maximum(
        jnp.dot(x, w1_ref[...], preferred_element_type=jnp.float32)
        + b1_ref[...], 0.0)
    h = jnp.maximum(
        jnp.dot(h, w2_ref[...], preferred_element_type=jnp.float32)
        + b2_ref[...], 0.0)
    dnn = jnp.maximum(
        jnp.dot(h, w3_ref[...], preferred_element_type=jnp.float32)
        + b3_ref[...], 0.0)

    xl = x
    for i in range(N_CROSS):
        alpha = jnp.dot(xl, cw_ref[:, i:i + 1],
                        preferred_element_type=jnp.float32)
        xl = (x * alpha + sc_ref[:, i:i + 1]) + xl

    cat = jnp.concatenate([xl, dnn], axis=1)
    logit = jnp.dot(cat, wo_ref[...],
                    preferred_element_type=jnp.float32) + sc_ref[:, 3:4]
    out_ref[...] = jax.nn.sigmoid(logit)


def _tc_dcn(inputs, emb, cw, b1, w1, w2, b2, w3, b3, wo, sc,
            interpret=False):
    grid = (B // _BT,)

    def full(shape):
        return pl.BlockSpec(shape, lambda i: tuple(0 for _ in shape))

    return pl.pallas_call(
        _dcn_block,
        grid=grid,
        in_specs=[
            pl.BlockSpec((_BT, N_DENSE + N_SPARSE), lambda i: (i, 0)),
            pl.BlockSpec((_BT, N_SPARSE * EMB), lambda i: (i, 0)),
            full(cw.shape),
            full(b1.shape),
            full(w1.shape),
            full(w2.shape),
            full(b2.shape),
            full(w3.shape),
            full(b3.shape),
            full(wo.shape),
            full(sc.shape),
        ],
        out_specs=pl.BlockSpec((_BT, 1), lambda i: (i, 0)),
        out_shape=jax.ShapeDtypeStruct((B, 1), jnp.float32),
        interpret=interpret,
    )(inputs, emb, cw, b1, w1, w2, b2, w3, b3, wo, sc)


def kernel(inputs, embed_tables, cross_w, cross_b, W1, b1, W2, b2, W3, b3,
           Wo, bo):
    idx = inputs[:, N_DENSE:].astype(jnp.int32)
    flat_idx = (idx + jnp.arange(N_SPARSE, dtype=jnp.int32)[None, :]
                * VOCAB).reshape(_N_ROWS // _IDXW, _IDXW)
    table_flat = embed_tables.reshape(N_SPARSE * VOCAB, EMB)

    emb = _sc_gather(table_flat, flat_idx).reshape(B, N_SPARSE * EMB)

    cw = jnp.concatenate([cross_w[0], cross_w[1], cross_w[2]], axis=1)
    sc = jnp.concatenate([cross_b.reshape(-1), bo.reshape(-1)]).reshape(1, 4)
    return _tc_dcn(inputs, emb, cw, b1.reshape(1, -1), W1,
                   W2, b2.reshape(1, -1), W3, b3.reshape(1, -1), Wo, sc)
